# Initial kernel scaffold; baseline (speedup 1.0000x reference)
import jax
import jax.numpy as jnp
from jax.experimental import pallas as pl


def kernel(species, senders, receivers, segment_ids, target_species, params):
    raise NotImplementedError("write your pallas kernel here")


# speedup vs baseline: 1.6324x; 1.6324x over previous
import functools

import jax
import jax.numpy as jnp
from jax import lax
from jax.experimental import pallas as pl
from jax.experimental.pallas import tpu as pltpu
from jax.experimental.pallas import tpu_sc as plsc

N_NODES = 10000
N_EDGES = 160000
NUM_GRAPHS = 64
LATENT = 128
NUM_ELEMENTS = 5
N_RADII = 64
N_CENTERS = 100

NC = 2
NS = 16
NW = NC * NS

CHUNK = 128
EDGES_PAD = 163840
CHUNKS_PER_W = EDGES_PAD // (NW * CHUNK)
EDGES_PER_W = EDGES_PAD // NW
N_PAD = 10240
ROWS_PER_SUB = N_PAD // NS

BN = 1000
BN2 = 1024
BE = 2048

_f32 = jnp.float32
_i32 = jnp.int32

def _sc_gather_body(nodes_hbm, send_hbm, recv_hbm, sf_hbm, rf_hbm,
                    sidx, ridx, sbuf, rbuf, ssem, rsem):
    wid = lax.axis_index("s") * NC + lax.axis_index("c")
    cbase = wid * CHUNKS_PER_W
    pltpu.sync_copy(send_hbm.at[pl.ds(cbase, CHUNKS_PER_W)], sidx)
    pltpu.sync_copy(recv_hbm.at[pl.ds(cbase, CHUNKS_PER_W)], ridx)

    def body(j, carry):
        cs = pltpu.async_copy(nodes_hbm.at[sidx.at[j]], sbuf, ssem)
        cr = pltpu.async_copy(nodes_hbm.at[ridx.at[j]], rbuf, rsem)
        cs.wait()
        cr.wait()
        row = wid * EDGES_PER_W + j * CHUNK
        pltpu.sync_copy(sbuf, sf_hbm.at[pl.ds(row, CHUNK)])
        pltpu.sync_copy(rbuf, rf_hbm.at[pl.ds(row, CHUNK)])
        return carry

    lax.fori_loop(0, CHUNKS_PER_W, body, 0)


def _sc_scatter_body(msg_hbm, recv_hbm, zeros_hbm, out_hbm, ridx, mbuf, acc):
    cid = lax.axis_index("c")
    sid = lax.axis_index("s")
    wid = sid * NC + cid
    pltpu.sync_copy(zeros_hbm, acc.at[pl.ds(sid * ROWS_PER_SUB, ROWS_PER_SUB)])
    plsc.subcore_barrier()
    pltpu.sync_copy(recv_hbm.at[pl.ds(wid * CHUNKS_PER_W, CHUNKS_PER_W)], ridx)

    def body(j, carry):
        row = wid * EDGES_PER_W + j * CHUNK
        pltpu.sync_copy(msg_hbm.at[pl.ds(row, CHUNK)], mbuf)
        pltpu.sync_copy(mbuf, acc.at[ridx.at[j]], add=True)
        return carry

    lax.fori_loop(0, CHUNKS_PER_W, body, 0)
    plsc.subcore_barrier()
    pltpu.sync_copy(
        acc.at[pl.ds(sid * ROWS_PER_SUB, ROWS_PER_SUB)],
        out_hbm.at[cid].at[pl.ds(sid * ROWS_PER_SUB, ROWS_PER_SUB)],
    )


@functools.cache
def _sc_kernels():
    mesh = plsc.VectorSubcoreMesh(
        core_axis_name="c", subcore_axis_name="s",
        num_cores=NC, num_subcores=NS)
    gather = pl.kernel(
        _sc_gather_body,
        out_type=(
            jax.ShapeDtypeStruct((EDGES_PAD, LATENT), _f32),
            jax.ShapeDtypeStruct((EDGES_PAD, LATENT), _f32),
        ),
        mesh=mesh,
        scratch_types=[
            pltpu.VMEM((CHUNKS_PER_W, CHUNK), _i32),
            pltpu.VMEM((CHUNKS_PER_W, CHUNK), _i32),
            pltpu.VMEM((CHUNK, LATENT), _f32),
            pltpu.VMEM((CHUNK, LATENT), _f32),
            pltpu.SemaphoreType.DMA,
            pltpu.SemaphoreType.DMA,
        ],
    )
    scatter = pl.kernel(
        _sc_scatter_body,
        out_type=jax.ShapeDtypeStruct((NC, N_PAD, LATENT), _f32),
        mesh=mesh,
        scratch_types=[
            pltpu.VMEM((CHUNKS_PER_W, CHUNK), _i32),
            pltpu.VMEM((CHUNK, LATENT), _f32),
            pltpu.VMEM_SHARED((N_PAD, LATENT), _f32),
        ],
    )
    return gather, scatter


def _ssp(x):
    return (jnp.maximum(x, 0.0) + jnp.log1p(jnp.exp(-jnp.abs(x)))
            - 0.6931471805599453)


def _embed_init_body(spec_ref, se_ref, out_ref):
    spec = spec_ref[...]
    acc = jnp.zeros((BN2, LATENT), _f32)
    for e in range(NUM_ELEMENTS):
        m = (spec == e).astype(_f32)
        acc = acc + m * se_ref[e:e + 1, :]
    out_ref[...] = acc


def _edge_body(sf_ref, rf_ref, cen_ref, w1_ref, b1_ref, w2_ref, b2_ref,
               out_ref):
    sf = sf_ref[...]
    rf = rf_ref[...]
    diff = sf - rf
    d2 = jnp.sum(diff * diff, axis=1, keepdims=True)
    d = jnp.sqrt(d2 + 1e-12)
    t = d - cen_ref[...]
    emb = jnp.exp(-(t * t))
    h = _ssp(jnp.dot(emb, w1_ref[...], preferred_element_type=_f32)
             + b1_ref[...])
    filt = jnp.dot(h, w2_ref[...], preferred_element_type=_f32) + b2_ref[...]
    msg = filt * sf
    eid = (pl.program_id(0) * BE
           + lax.broadcasted_iota(_i32, (BE, 1), 0))
    out_ref[...] = jnp.where(eid < N_EDGES, msg, 0.0)


def _node_body(acc_ref, v1_ref, c1_ref, v2_ref, c2_ref, v3_ref, c3_ref,
               out_ref):
    x = acc_ref[0] + acc_ref[1]
    h = _ssp(jnp.dot(x, v1_ref[...], preferred_element_type=_f32)
             + c1_ref[...])
    h = _ssp(jnp.dot(h, v2_ref[...], preferred_element_type=_f32)
             + c2_ref[...])
    out_ref[...] = (jnp.dot(h, v3_ref[...], preferred_element_type=_f32)
                    + c3_ref[...])


def _final_body(x_ref, se_ref, t1_ref, u1_ref, t2_ref, u2_ref, t3_ref,
                u3_ref, d1_ref, e1_ref, d2_ref, e2_ref, d3_ref, e3_ref,
                seg_ref, tsp_ref, tsl_ref, dl_ref):
    i = pl.program_id(0)
    x = x_ref[...]
    per = jnp.zeros((BN, 8), _f32)
    col = lax.broadcasted_iota(_i32, (1, 8), 1)
    for e in range(NUM_ELEMENTS):
        xe = x * se_ref[e:e + 1, :]
        h = _ssp(jnp.dot(xe, t1_ref[...], preferred_element_type=_f32)
                 + u1_ref[...])
        h = _ssp(jnp.dot(h, t2_ref[...], preferred_element_type=_f32)
                 + u2_ref[...])
        pe = jnp.dot(h, t3_ref[...], preferred_element_type=_f32) + u3_ref[...]
        per = per + pe * (col == e).astype(_f32)
    seg = seg_ref[...]
    onehot = (seg == lax.broadcasted_iota(_i32, (BN, NUM_GRAPHS), 1)
              ).astype(_f32)
    part = lax.dot_general(onehot, per, (((0,), (0,)), ((), ())),
                           preferred_element_type=_f32)

    @pl.when(i == 0)
    def _():
        tsl_ref[...] = jnp.zeros_like(tsl_ref)

    tsl_ref[...] += part

    tsp = tsp_ref[...]
    tse = jnp.zeros((BN, LATENT), _f32)
    for e in range(NUM_ELEMENTS):
        m = (tsp == e).astype(_f32)
        tse = tse + m * se_ref[e:e + 1, :]
    y = x * tse
    h = _ssp(jnp.dot(y, d1_ref[...], preferred_element_type=_f32)
             + e1_ref[...])
    h = _ssp(jnp.dot(h, d2_ref[...], preferred_element_type=_f32)
             + e2_ref[...])
    dl_ref[...] = (jnp.dot(h, d3_ref[...], preferred_element_type=_f32)
                   + e3_ref[...])


def _full(shape):
    return pl.BlockSpec(shape, lambda i: tuple(0 for _ in shape))


_embed_init = pl.pallas_call(
    _embed_init_body,
    grid=(N_PAD // BN2,),
    in_specs=[
        pl.BlockSpec((BN2, 1), lambda i: (i, 0)),
        _full((NUM_ELEMENTS, LATENT)),
    ],
    out_specs=pl.BlockSpec((BN2, LATENT), lambda i: (i, 0)),
    out_shape=jax.ShapeDtypeStruct((N_PAD, LATENT), _f32),
)

_edge_dense = pl.pallas_call(
    _edge_body,
    grid=(EDGES_PAD // BE,),
    in_specs=[
        pl.BlockSpec((BE, LATENT), lambda i: (i, 0)),
        pl.BlockSpec((BE, LATENT), lambda i: (i, 0)),
        _full((1, LATENT)),
        _full((LATENT, LATENT)),
        _full((1, LATENT)),
        _full((LATENT, LATENT)),
        _full((1, LATENT)),
    ],
    out_specs=pl.BlockSpec((BE, LATENT), lambda i: (i, 0)),
    out_shape=jax.ShapeDtypeStruct((EDGES_PAD, LATENT), _f32),
)

_node_mlp = pl.pallas_call(
    _node_body,
    grid=(N_PAD // BN2,),
    in_specs=[
        pl.BlockSpec((NC, BN2, LATENT), lambda i: (0, i, 0)),
        _full((LATENT, LATENT)),
        _full((1, LATENT)),
        _full((LATENT, LATENT)),
        _full((1, LATENT)),
        _full((LATENT, LATENT)),
        _full((1, LATENT)),
    ],
    out_specs=pl.BlockSpec((BN2, LATENT), lambda i: (i, 0)),
    out_shape=jax.ShapeDtypeStruct((N_PAD, LATENT), _f32),
)

_final_heads = pl.pallas_call(
    _final_body,
    grid=(N_NODES // BN,),
    in_specs=[
        pl.BlockSpec((BN, LATENT), lambda i: (i, 0)),
        _full((NUM_ELEMENTS, LATENT)),
        _full((LATENT, LATENT)),
        _full((1, LATENT)),
        _full((LATENT, LATENT)),
        _full((1, LATENT)),
        _full((LATENT, 8)),
        _full((1, 8)),
        _full((LATENT, LATENT)),
        _full((1, LATENT)),
        _full((LATENT, LATENT)),
        _full((1, LATENT)),
        _full((LATENT, N_RADII)),
        _full((1, N_RADII)),
        pl.BlockSpec((BN, 1), lambda i: (i, 0)),
        pl.BlockSpec((BN, 1), lambda i: (i, 0)),
    ],
    out_specs=[
        _full((NUM_GRAPHS, 8)),
        pl.BlockSpec((BN, N_RADII), lambda i: (i, 0)),
    ],
    out_shape=[
        jax.ShapeDtypeStruct((NUM_GRAPHS, 8), _f32),
        jax.ShapeDtypeStruct((N_NODES, N_RADII), _f32),
    ],
)


def kernel(species, senders, receivers, segment_ids, target_species, params):
    spec2 = jnp.pad(species.astype(_i32), (0, N_PAD - N_NODES),
                    constant_values=NUM_ELEMENTS).reshape(N_PAD, 1)
    seg2 = segment_ids.astype(_i32).reshape(N_NODES, 1)
    tsp2 = target_species.astype(_i32).reshape(N_NODES, 1)
    send_p = jnp.pad(senders.astype(_i32), (0, EDGES_PAD - N_EDGES)).reshape(
        EDGES_PAD // CHUNK, CHUNK)
    recv_p = jnp.pad(receivers.astype(_i32), (0, EDGES_PAD - N_EDGES)).reshape(
        EDGES_PAD // CHUNK, CHUNK)
    zeros = jnp.zeros((ROWS_PER_SUB, LATENT), _f32)
    se = params["species_embed"].astype(_f32)

    centers = jnp.linspace(0.0, 10.0, N_CENTERS).astype(_f32)
    cen_p = jnp.concatenate(
        [centers, jnp.full((LATENT - N_CENTERS,), 1e4, _f32)]).reshape(1, LATENT)

    def row(b):
        return b.astype(_f32).reshape(1, -1)

    _sc_gather, _sc_scatter = _sc_kernels()
    nodes = _embed_init(spec2, se)
    for blk in params["blocks"]:
        (w1, b1), (w2, b2) = blk["cfconv"]
        w1p = jnp.pad(w1.astype(_f32), ((0, LATENT - N_CENTERS), (0, 0)))
        sf, rf = _sc_gather(nodes, send_p, recv_p)
        msgs = _edge_dense(sf, rf, cen_p, w1p, row(b1), w2.astype(_f32),
                           row(b2))
        parts = _sc_scatter(msgs, recv_p, zeros)
        (v1, c1), (v2, c2), (v3, c3) = blk["node"]
        nodes = _node_mlp(parts, v1.astype(_f32), row(c1), v2.astype(_f32),
                          row(c2), v3.astype(_f32), row(c3))

    nodes = nodes[:N_NODES]
    (t1, u1), (t2, u2), (t3, u3) = params["target_mlp"]
    (d1, e1), (d2, e2), (d3, e3) = params["dist_mlp"]
    t3p = jnp.pad(t3.astype(_f32), ((0, 0), (0, 8 - NUM_ELEMENTS)))
    u3p = jnp.pad(u3.astype(_f32), (0, 8 - NUM_ELEMENTS)).reshape(1, 8)
    tslp, dl = _final_heads(
        nodes, se, t1.astype(_f32), row(u1), t2.astype(_f32), row(u2),
        t3p, u3p, d1.astype(_f32), row(e1), d2.astype(_f32), row(e2),
        d3.astype(_f32), row(e3), seg2, tsp2)
    return tslp[:, :NUM_ELEMENTS], dl


# speedup vs baseline: 1.8690x; 1.1449x over previous
import functools

import jax
import jax.numpy as jnp
from jax import lax
from jax.experimental import pallas as pl
from jax.experimental.pallas import tpu as pltpu
from jax.experimental.pallas import tpu_sc as plsc

N_NODES = 10000
N_EDGES = 160000
NUM_GRAPHS = 64
LATENT = 128
NUM_ELEMENTS = 5
N_RADII = 64
N_CENTERS = 100

NC = 2
NS = 16
NW = NC * NS

CHUNK = 128
EDGES_PAD = 163840
CHUNKS_PER_W = EDGES_PAD // (NW * CHUNK)
EDGES_PER_W = EDGES_PAD // NW
N_PAD = 10240
ROWS_PER_SUB = N_PAD // NS

BN = 1000
BN2 = 1024
BE = 2048

_f32 = jnp.float32
_i32 = jnp.int32

def _sc_gather_body(nodes_hbm, send_hbm, recv_hbm, sf_hbm, rf_hbm,
                    sidx, ridx, sbuf, rbuf,
                    gs0, gs1, gr0, gr1, ws0, ws1, wr0, wr1):
    wid = lax.axis_index("s") * NC + lax.axis_index("c")
    cbase = wid * CHUNKS_PER_W
    pltpu.sync_copy(send_hbm.at[pl.ds(cbase, CHUNKS_PER_W)], sidx)
    pltpu.sync_copy(recv_hbm.at[pl.ds(cbase, CHUNKS_PER_W)], ridx)
    base = wid * EDGES_PER_W
    gsem = (gs0, gs1)
    grsem = (gr0, gr1)
    wsem = (ws0, ws1)
    wrsem = (wr0, wr1)

    def start_gather(j, b):
        pltpu.async_copy(nodes_hbm.at[sidx.at[j]], sbuf.at[b], gsem[b])
        pltpu.async_copy(nodes_hbm.at[ridx.at[j]], rbuf.at[b], grsem[b])

    def wait_gather(b):
        pltpu.make_async_copy(nodes_hbm.at[sidx.at[0]], sbuf.at[b],
                              gsem[b]).wait()
        pltpu.make_async_copy(nodes_hbm.at[ridx.at[0]], rbuf.at[b],
                              grsem[b]).wait()

    def start_write(j, b):
        row = base + j * CHUNK
        pltpu.async_copy(sbuf.at[b], sf_hbm.at[pl.ds(row, CHUNK)], wsem[b])
        pltpu.async_copy(rbuf.at[b], rf_hbm.at[pl.ds(row, CHUNK)], wrsem[b])

    def wait_write(b):
        pltpu.make_async_copy(sbuf.at[b], sf_hbm.at[pl.ds(0, CHUNK)],
                              wsem[b]).wait()
        pltpu.make_async_copy(rbuf.at[b], rf_hbm.at[pl.ds(0, CHUNK)],
                              wrsem[b]).wait()

    start_gather(0, 0)
    start_gather(1, 1)

    def pair(i, carry):
        for b in (0, 1):
            j = 2 * i + b
            wait_gather(b)
            start_write(j, b)

            @pl.when(j + 2 < CHUNKS_PER_W)
            def _():
                wait_write(b)
                start_gather(j + 2, b)
        return carry

    lax.fori_loop(0, CHUNKS_PER_W // 2, pair, 0)
    wait_write(0)
    wait_write(1)


def _sc_scatter_body(msg_hbm, recv_hbm, zeros_hbm, out_hbm, ridx, mbuf, acc,
                     rs0, rs1, as0, as1):
    cid = lax.axis_index("c")
    sid = lax.axis_index("s")
    wid = sid * NC + cid
    pltpu.sync_copy(zeros_hbm, acc.at[pl.ds(sid * ROWS_PER_SUB, ROWS_PER_SUB)])
    plsc.subcore_barrier()
    pltpu.sync_copy(recv_hbm.at[pl.ds(wid * CHUNKS_PER_W, CHUNKS_PER_W)], ridx)
    base = wid * EDGES_PER_W
    rsem = (rs0, rs1)
    asem = (as0, as1)

    rsem = (rs0, rs1)
    del as0, as1

    def start_read(j, b):
        pltpu.async_copy(msg_hbm.at[pl.ds(base + j * CHUNK, CHUNK)],
                         mbuf.at[b], rsem[b])

    def wait_read(b):
        pltpu.make_async_copy(msg_hbm.at[pl.ds(0, CHUNK)], mbuf.at[b],
                              rsem[b]).wait()

    start_read(0, 0)

    def pair(i, carry):
        for b in (0, 1):
            j = 2 * i + b

            @pl.when(j + 1 < CHUNKS_PER_W)
            def _():
                start_read(j + 1, 1 - b)

            wait_read(b)
            pltpu.sync_copy(mbuf.at[b], acc.at[ridx.at[j]], add=True)
        return carry

    lax.fori_loop(0, CHUNKS_PER_W // 2, pair, 0)
    plsc.subcore_barrier()
    pltpu.sync_copy(
        acc.at[pl.ds(sid * ROWS_PER_SUB, ROWS_PER_SUB)],
        out_hbm.at[cid].at[pl.ds(sid * ROWS_PER_SUB, ROWS_PER_SUB)],
    )


@functools.cache
def _sc_kernels():
    mesh = plsc.VectorSubcoreMesh(
        core_axis_name="c", subcore_axis_name="s",
        num_cores=NC, num_subcores=NS)
    gather = pl.kernel(
        _sc_gather_body,
        out_type=(
            jax.ShapeDtypeStruct((EDGES_PAD, LATENT), _f32),
            jax.ShapeDtypeStruct((EDGES_PAD, LATENT), _f32),
        ),
        mesh=mesh,
        scratch_types=[
            pltpu.VMEM((CHUNKS_PER_W, CHUNK), _i32),
            pltpu.VMEM((CHUNKS_PER_W, CHUNK), _i32),
            pltpu.VMEM((2, CHUNK, LATENT), _f32),
            pltpu.VMEM((2, CHUNK, LATENT), _f32),
        ] + [pltpu.SemaphoreType.DMA] * 8,
    )
    scatter = pl.kernel(
        _sc_scatter_body,
        out_type=jax.ShapeDtypeStruct((NC, N_PAD, LATENT), _f32),
        mesh=mesh,
        scratch_types=[
            pltpu.VMEM((CHUNKS_PER_W, CHUNK), _i32),
            pltpu.VMEM((2, CHUNK, LATENT), _f32),
            pltpu.VMEM_SHARED((N_PAD, LATENT), _f32),
        ] + [pltpu.SemaphoreType.DMA] * 4,
    )
    return gather, scatter


def _ssp(x):
    return (jnp.maximum(x, 0.0) + jnp.log1p(jnp.exp(-jnp.abs(x)))
            - 0.6931471805599453)


def _embed_init_body(spec_ref, se_ref, out_ref):
    spec = spec_ref[...]
    acc = jnp.zeros((BN2, LATENT), _f32)
    for e in range(NUM_ELEMENTS):
        m = (spec == e).astype(_f32)
        acc = acc + m * se_ref[e:e + 1, :]
    out_ref[...] = acc


def _edge_body(sf_ref, rf_ref, cen_ref, w1_ref, b1_ref, w2_ref, b2_ref,
               out_ref):
    sf = sf_ref[...]
    rf = rf_ref[...]
    diff = sf - rf
    d2 = jnp.sum(diff * diff, axis=1, keepdims=True)
    d = jnp.sqrt(d2 + 1e-12)
    t = d - cen_ref[...]
    emb = jnp.exp(-(t * t))
    h = _ssp(jnp.dot(emb, w1_ref[...], preferred_element_type=_f32)
             + b1_ref[...])
    filt = jnp.dot(h, w2_ref[...], preferred_element_type=_f32) + b2_ref[...]
    msg = filt * sf
    eid = (pl.program_id(0) * BE
           + lax.broadcasted_iota(_i32, (BE, 1), 0))
    out_ref[...] = jnp.where(eid < N_EDGES, msg, 0.0)


def _node_body(acc_ref, v1_ref, c1_ref, v2_ref, c2_ref, v3_ref, c3_ref,
               out_ref):
    x = acc_ref[0] + acc_ref[1]
    h = _ssp(jnp.dot(x, v1_ref[...], preferred_element_type=_f32)
             + c1_ref[...])
    h = _ssp(jnp.dot(h, v2_ref[...], preferred_element_type=_f32)
             + c2_ref[...])
    out_ref[...] = (jnp.dot(h, v3_ref[...], preferred_element_type=_f32)
                    + c3_ref[...])


def _final_body(x_ref, se_ref, t1_ref, u1_ref, t2_ref, u2_ref, t3_ref,
                u3_ref, d1_ref, e1_ref, d2_ref, e2_ref, d3_ref, e3_ref,
                seg_ref, tsp_ref, tsl_ref, dl_ref):
    i = pl.program_id(0)
    x = x_ref[...]
    per = jnp.zeros((BN, 8), _f32)
    col = lax.broadcasted_iota(_i32, (1, 8), 1)
    for e in range(NUM_ELEMENTS):
        xe = x * se_ref[e:e + 1, :]
        h = _ssp(jnp.dot(xe, t1_ref[...], preferred_element_type=_f32)
                 + u1_ref[...])
        h = _ssp(jnp.dot(h, t2_ref[...], preferred_element_type=_f32)
                 + u2_ref[...])
        pe = jnp.dot(h, t3_ref[...], preferred_element_type=_f32) + u3_ref[...]
        per = per + pe * (col == e).astype(_f32)
    seg = seg_ref[...]
    onehot = (seg == lax.broadcasted_iota(_i32, (BN, NUM_GRAPHS), 1)
              ).astype(_f32)
    part = lax.dot_general(onehot, per, (((0,), (0,)), ((), ())),
                           preferred_element_type=_f32)

    @pl.when(i == 0)
    def _():
        tsl_ref[...] = jnp.zeros_like(tsl_ref)

    tsl_ref[...] += part

    tsp = tsp_ref[...]
    tse = jnp.zeros((BN, LATENT), _f32)
    for e in range(NUM_ELEMENTS):
        m = (tsp == e).astype(_f32)
        tse = tse + m * se_ref[e:e + 1, :]
    y = x * tse
    h = _ssp(jnp.dot(y, d1_ref[...], preferred_element_type=_f32)
             + e1_ref[...])
    h = _ssp(jnp.dot(h, d2_ref[...], preferred_element_type=_f32)
             + e2_ref[...])
    dl_ref[...] = (jnp.dot(h, d3_ref[...], preferred_element_type=_f32)
                   + e3_ref[...])


def _full(shape):
    return pl.BlockSpec(shape, lambda i: tuple(0 for _ in shape))


_embed_init = pl.pallas_call(
    _embed_init_body,
    grid=(N_PAD // BN2,),
    in_specs=[
        pl.BlockSpec((BN2, 1), lambda i: (i, 0)),
        _full((NUM_ELEMENTS, LATENT)),
    ],
    out_specs=pl.BlockSpec((BN2, LATENT), lambda i: (i, 0)),
    out_shape=jax.ShapeDtypeStruct((N_PAD, LATENT), _f32),
)

_edge_dense = pl.pallas_call(
    _edge_body,
    grid=(EDGES_PAD // BE,),
    in_specs=[
        pl.BlockSpec((BE, LATENT), lambda i: (i, 0)),
        pl.BlockSpec((BE, LATENT), lambda i: (i, 0)),
        _full((1, LATENT)),
        _full((LATENT, LATENT)),
        _full((1, LATENT)),
        _full((LATENT, LATENT)),
        _full((1, LATENT)),
    ],
    out_specs=pl.BlockSpec((BE, LATENT), lambda i: (i, 0)),
    out_shape=jax.ShapeDtypeStruct((EDGES_PAD, LATENT), _f32),
)

_node_mlp = pl.pallas_call(
    _node_body,
    grid=(N_PAD // BN2,),
    in_specs=[
        pl.BlockSpec((NC, BN2, LATENT), lambda i: (0, i, 0)),
        _full((LATENT, LATENT)),
        _full((1, LATENT)),
        _full((LATENT, LATENT)),
        _full((1, LATENT)),
        _full((LATENT, LATENT)),
        _full((1, LATENT)),
    ],
    out_specs=pl.BlockSpec((BN2, LATENT), lambda i: (i, 0)),
    out_shape=jax.ShapeDtypeStruct((N_PAD, LATENT), _f32),
)

_final_heads = pl.pallas_call(
    _final_body,
    grid=(N_NODES // BN,),
    in_specs=[
        pl.BlockSpec((BN, LATENT), lambda i: (i, 0)),
        _full((NUM_ELEMENTS, LATENT)),
        _full((LATENT, LATENT)),
        _full((1, LATENT)),
        _full((LATENT, LATENT)),
        _full((1, LATENT)),
        _full((LATENT, 8)),
        _full((1, 8)),
        _full((LATENT, LATENT)),
        _full((1, LATENT)),
        _full((LATENT, LATENT)),
        _full((1, LATENT)),
        _full((LATENT, N_RADII)),
        _full((1, N_RADII)),
        pl.BlockSpec((BN, 1), lambda i: (i, 0)),
        pl.BlockSpec((BN, 1), lambda i: (i, 0)),
    ],
    out_specs=[
        _full((NUM_GRAPHS, 8)),
        pl.BlockSpec((BN, N_RADII), lambda i: (i, 0)),
    ],
    out_shape=[
        jax.ShapeDtypeStruct((NUM_GRAPHS, 8), _f32),
        jax.ShapeDtypeStruct((N_NODES, N_RADII), _f32),
    ],
)


def kernel(species, senders, receivers, segment_ids, target_species, params):
    spec2 = jnp.pad(species.astype(_i32), (0, N_PAD - N_NODES),
                    constant_values=NUM_ELEMENTS).reshape(N_PAD, 1)
    seg2 = segment_ids.astype(_i32).reshape(N_NODES, 1)
    tsp2 = target_species.astype(_i32).reshape(N_NODES, 1)
    send_p = jnp.pad(senders.astype(_i32), (0, EDGES_PAD - N_EDGES)).reshape(
        EDGES_PAD // CHUNK, CHUNK)
    recv_p = jnp.pad(receivers.astype(_i32), (0, EDGES_PAD - N_EDGES)).reshape(
        EDGES_PAD // CHUNK, CHUNK)
    zeros = jnp.zeros((ROWS_PER_SUB, LATENT), _f32)
    se = params["species_embed"].astype(_f32)

    centers = jnp.linspace(0.0, 10.0, N_CENTERS).astype(_f32)
    cen_p = jnp.concatenate(
        [centers, jnp.full((LATENT - N_CENTERS,), 1e4, _f32)]).reshape(1, LATENT)

    def row(b):
        return b.astype(_f32).reshape(1, -1)

    _sc_gather, _sc_scatter = _sc_kernels()
    nodes = _embed_init(spec2, se)
    for blk in params["blocks"]:
        (w1, b1), (w2, b2) = blk["cfconv"]
        w1p = jnp.pad(w1.astype(_f32), ((0, LATENT - N_CENTERS), (0, 0)))
        sf, rf = _sc_gather(nodes, send_p, recv_p)
        msgs = _edge_dense(sf, rf, cen_p, w1p, row(b1), w2.astype(_f32),
                           row(b2))
        parts = _sc_scatter(msgs, recv_p, zeros)
        (v1, c1), (v2, c2), (v3, c3) = blk["node"]
        nodes = _node_mlp(parts, v1.astype(_f32), row(c1), v2.astype(_f32),
                          row(c2), v3.astype(_f32), row(c3))

    nodes = nodes[:N_NODES]
    (t1, u1), (t2, u2), (t3, u3) = params["target_mlp"]
    (d1, e1), (d2, e2), (d3, e3) = params["dist_mlp"]
    t3p = jnp.pad(t3.astype(_f32), ((0, 0), (0, 8 - NUM_ELEMENTS)))
    u3p = jnp.pad(u3.astype(_f32), (0, 8 - NUM_ELEMENTS)).reshape(1, 8)
    tslp, dl = _final_heads(
        nodes, se, t1.astype(_f32), row(u1), t2.astype(_f32), row(u2),
        t3p, u3p, d1.astype(_f32), row(e1), d2.astype(_f32), row(e2),
        d3.astype(_f32), row(e3), seg2, tsp2)
    return tslp[:, :NUM_ELEMENTS], dl


# speedup vs baseline: 3.9575x; 2.1175x over previous
import functools

import jax
import jax.numpy as jnp
from jax import lax
from jax.experimental import pallas as pl
from jax.experimental.pallas import tpu as pltpu
from jax.experimental.pallas import tpu_sc as plsc

N_NODES = 10000
N_EDGES = 160000
NUM_GRAPHS = 64
LATENT = 128
NUM_ELEMENTS = 5
N_RADII = 64
N_CENTERS = 100

NC = 2
NS = 16
NW = NC * NS

CHUNK = 128
EDGES_PAD = 163840
CHUNKS_PER_W = EDGES_PAD // (NW * CHUNK)
EDGES_PER_W = EDGES_PAD // NW
GCH = 64
GCHUNKS_PER_W = EDGES_PAD // (NW * GCH)
N_PAD = 10240
ROWS_PER_SUB = N_PAD // NS

BN = 1000
BN2 = 1024
BE = 2048

_f32 = jnp.float32
_i32 = jnp.int32

def _sc_gather_body(nodes_hbm, send_hbm, recv_hbm, sf_hbm, rf_hbm,
                    sidx, ridx, sbuf, rbuf, table,
                    ws0, ws1, wr0, wr1):
    cid = lax.axis_index("c")
    sid = lax.axis_index("s")
    wid = sid * NC + cid
    pltpu.sync_copy(nodes_hbm.at[pl.ds(sid * ROWS_PER_SUB, ROWS_PER_SUB)],
                    table.at[pl.ds(sid * ROWS_PER_SUB, ROWS_PER_SUB)])
    base = wid * EDGES_PER_W
    pltpu.sync_copy(send_hbm.at[pl.ds(base, EDGES_PER_W)], sidx)
    pltpu.sync_copy(recv_hbm.at[pl.ds(base, EDGES_PER_W)], ridx)
    plsc.subcore_barrier()
    wsem = (ws0, ws1)
    wrsem = (wr0, wr1)

    def start_write(j, b):
        row = base + j * GCH
        pltpu.async_copy(sbuf.at[b], sf_hbm.at[pl.ds(row, GCH)], wsem[b])
        pltpu.async_copy(rbuf.at[b], rf_hbm.at[pl.ds(row, GCH)], wrsem[b])

    def wait_write(b):
        pltpu.make_async_copy(sbuf.at[b], sf_hbm.at[pl.ds(0, GCH)],
                              wsem[b]).wait()
        pltpu.make_async_copy(rbuf.at[b], rf_hbm.at[pl.ds(0, GCH)],
                              wrsem[b]).wait()

    def pair(i, carry):
        for b in (0, 1):
            j = 2 * i + b

            @pl.when(j >= 2)
            def _():
                wait_write(b)

            pltpu.sync_copy(table.at[sidx.at[pl.ds(j * GCH, GCH)]],
                            sbuf.at[b])
            pltpu.sync_copy(table.at[ridx.at[pl.ds(j * GCH, GCH)]],
                            rbuf.at[b])
            start_write(j, b)
        return carry

    lax.fori_loop(0, GCHUNKS_PER_W // 2, pair, 0)
    wait_write(0)
    wait_write(1)


def _sc_scatter_body(msg_hbm, recv_hbm, zeros_hbm, out_hbm, ridx, mbuf, acc,
                     rs0, rs1, as0, as1):
    cid = lax.axis_index("c")
    sid = lax.axis_index("s")
    wid = sid * NC + cid
    pltpu.sync_copy(zeros_hbm, acc.at[pl.ds(sid * ROWS_PER_SUB, ROWS_PER_SUB)])
    plsc.subcore_barrier()
    pltpu.sync_copy(recv_hbm.at[pl.ds(wid * CHUNKS_PER_W, CHUNKS_PER_W)], ridx)
    base = wid * EDGES_PER_W
    rsem = (rs0, rs1)
    asem = (as0, as1)

    rsem = (rs0, rs1)
    del as0, as1

    def start_read(j, b):
        pltpu.async_copy(msg_hbm.at[pl.ds(base + j * CHUNK, CHUNK)],
                         mbuf.at[b], rsem[b])

    def wait_read(b):
        pltpu.make_async_copy(msg_hbm.at[pl.ds(0, CHUNK)], mbuf.at[b],
                              rsem[b]).wait()

    start_read(0, 0)

    def pair(i, carry):
        for b in (0, 1):
            j = 2 * i + b

            @pl.when(j + 1 < CHUNKS_PER_W)
            def _():
                start_read(j + 1, 1 - b)

            wait_read(b)
            pltpu.sync_copy(mbuf.at[b], acc.at[ridx.at[j]], add=True)
        return carry

    lax.fori_loop(0, CHUNKS_PER_W // 2, pair, 0)
    plsc.subcore_barrier()
    pltpu.sync_copy(
        acc.at[pl.ds(sid * ROWS_PER_SUB, ROWS_PER_SUB)],
        out_hbm.at[cid].at[pl.ds(sid * ROWS_PER_SUB, ROWS_PER_SUB)],
    )


@functools.cache
def _sc_kernels():
    mesh = plsc.VectorSubcoreMesh(
        core_axis_name="c", subcore_axis_name="s",
        num_cores=NC, num_subcores=NS)
    gather = pl.kernel(
        _sc_gather_body,
        out_type=(
            jax.ShapeDtypeStruct((EDGES_PAD, LATENT), _f32),
            jax.ShapeDtypeStruct((EDGES_PAD, LATENT), _f32),
        ),
        mesh=mesh,
        scratch_types=[
            pltpu.VMEM((EDGES_PER_W,), _i32),
            pltpu.VMEM((EDGES_PER_W,), _i32),
            pltpu.VMEM((2, GCH, LATENT), _f32),
            pltpu.VMEM((2, GCH, LATENT), _f32),
            pltpu.VMEM_SHARED((N_PAD, LATENT), _f32),
        ] + [pltpu.SemaphoreType.DMA] * 4,
    )
    scatter = pl.kernel(
        _sc_scatter_body,
        out_type=jax.ShapeDtypeStruct((NC, N_PAD, LATENT), _f32),
        mesh=mesh,
        scratch_types=[
            pltpu.VMEM((CHUNKS_PER_W, CHUNK), _i32),
            pltpu.VMEM((2, CHUNK, LATENT), _f32),
            pltpu.VMEM_SHARED((N_PAD, LATENT), _f32),
        ] + [pltpu.SemaphoreType.DMA] * 4,
    )
    return gather, scatter


def _ssp(x):
    return (jnp.maximum(x, 0.0) + jnp.log1p(jnp.exp(-jnp.abs(x)))
            - 0.6931471805599453)


def _embed_init_body(spec_ref, se_ref, out_ref):
    spec = spec_ref[...]
    acc = jnp.zeros((BN2, LATENT), _f32)
    for e in range(NUM_ELEMENTS):
        m = (spec == e).astype(_f32)
        acc = acc + m * se_ref[e:e + 1, :]
    out_ref[...] = acc


def _edge_body(sf_ref, rf_ref, cen_ref, w1_ref, b1_ref, w2_ref, b2_ref,
               out_ref):
    sf = sf_ref[...]
    rf = rf_ref[...]
    diff = sf - rf
    d2 = jnp.sum(diff * diff, axis=1, keepdims=True)
    d = jnp.sqrt(d2 + 1e-12)
    t = d - cen_ref[...]
    emb = jnp.exp(-(t * t))
    h = _ssp(jnp.dot(emb, w1_ref[...], preferred_element_type=_f32)
             + b1_ref[...])
    filt = jnp.dot(h, w2_ref[...], preferred_element_type=_f32) + b2_ref[...]
    msg = filt * sf
    eid = (pl.program_id(0) * BE
           + lax.broadcasted_iota(_i32, (BE, 1), 0))
    out_ref[...] = jnp.where(eid < N_EDGES, msg, 0.0)


def _node_body(acc_ref, v1_ref, c1_ref, v2_ref, c2_ref, v3_ref, c3_ref,
               out_ref):
    x = acc_ref[0] + acc_ref[1]
    h = _ssp(jnp.dot(x, v1_ref[...], preferred_element_type=_f32)
             + c1_ref[...])
    h = _ssp(jnp.dot(h, v2_ref[...], preferred_element_type=_f32)
             + c2_ref[...])
    out_ref[...] = (jnp.dot(h, v3_ref[...], preferred_element_type=_f32)
                    + c3_ref[...])


def _final_body(x_ref, se_ref, t1_ref, u1_ref, t2_ref, u2_ref, t3_ref,
                u3_ref, d1_ref, e1_ref, d2_ref, e2_ref, d3_ref, e3_ref,
                seg_ref, tsp_ref, tsl_ref, dl_ref):
    i = pl.program_id(0)
    x = x_ref[...]
    per = jnp.zeros((BN, 8), _f32)
    col = lax.broadcasted_iota(_i32, (1, 8), 1)
    for e in range(NUM_ELEMENTS):
        xe = x * se_ref[e:e + 1, :]
        h = _ssp(jnp.dot(xe, t1_ref[...], preferred_element_type=_f32)
                 + u1_ref[...])
        h = _ssp(jnp.dot(h, t2_ref[...], preferred_element_type=_f32)
                 + u2_ref[...])
        pe = jnp.dot(h, t3_ref[...], preferred_element_type=_f32) + u3_ref[...]
        per = per + pe * (col == e).astype(_f32)
    seg = seg_ref[...]
    onehot = (seg == lax.broadcasted_iota(_i32, (BN, NUM_GRAPHS), 1)
              ).astype(_f32)
    part = lax.dot_general(onehot, per, (((0,), (0,)), ((), ())),
                           preferred_element_type=_f32)

    @pl.when(i == 0)
    def _():
        tsl_ref[...] = jnp.zeros_like(tsl_ref)

    tsl_ref[...] += part

    tsp = tsp_ref[...]
    tse = jnp.zeros((BN, LATENT), _f32)
    for e in range(NUM_ELEMENTS):
        m = (tsp == e).astype(_f32)
        tse = tse + m * se_ref[e:e + 1, :]
    y = x * tse
    h = _ssp(jnp.dot(y, d1_ref[...], preferred_element_type=_f32)
             + e1_ref[...])
    h = _ssp(jnp.dot(h, d2_ref[...], preferred_element_type=_f32)
             + e2_ref[...])
    dl_ref[...] = (jnp.dot(h, d3_ref[...], preferred_element_type=_f32)
                   + e3_ref[...])


def _full(shape):
    return pl.BlockSpec(shape, lambda i: tuple(0 for _ in shape))


_embed_init = pl.pallas_call(
    _embed_init_body,
    grid=(N_PAD // BN2,),
    in_specs=[
        pl.BlockSpec((BN2, 1), lambda i: (i, 0)),
        _full((NUM_ELEMENTS, LATENT)),
    ],
    out_specs=pl.BlockSpec((BN2, LATENT), lambda i: (i, 0)),
    out_shape=jax.ShapeDtypeStruct((N_PAD, LATENT), _f32),
)

_edge_dense = pl.pallas_call(
    _edge_body,
    grid=(EDGES_PAD // BE,),
    in_specs=[
        pl.BlockSpec((BE, LATENT), lambda i: (i, 0)),
        pl.BlockSpec((BE, LATENT), lambda i: (i, 0)),
        _full((1, LATENT)),
        _full((LATENT, LATENT)),
        _full((1, LATENT)),
        _full((LATENT, LATENT)),
        _full((1, LATENT)),
    ],
    out_specs=pl.BlockSpec((BE, LATENT), lambda i: (i, 0)),
    out_shape=jax.ShapeDtypeStruct((EDGES_PAD, LATENT), _f32),
)

_node_mlp = pl.pallas_call(
    _node_body,
    grid=(N_PAD // BN2,),
    in_specs=[
        pl.BlockSpec((NC, BN2, LATENT), lambda i: (0, i, 0)),
        _full((LATENT, LATENT)),
        _full((1, LATENT)),
        _full((LATENT, LATENT)),
        _full((1, LATENT)),
        _full((LATENT, LATENT)),
        _full((1, LATENT)),
    ],
    out_specs=pl.BlockSpec((BN2, LATENT), lambda i: (i, 0)),
    out_shape=jax.ShapeDtypeStruct((N_PAD, LATENT), _f32),
)

_final_heads = pl.pallas_call(
    _final_body,
    grid=(N_NODES // BN,),
    in_specs=[
        pl.BlockSpec((BN, LATENT), lambda i: (i, 0)),
        _full((NUM_ELEMENTS, LATENT)),
        _full((LATENT, LATENT)),
        _full((1, LATENT)),
        _full((LATENT, LATENT)),
        _full((1, LATENT)),
        _full((LATENT, 8)),
        _full((1, 8)),
        _full((LATENT, LATENT)),
        _full((1, LATENT)),
        _full((LATENT, LATENT)),
        _full((1, LATENT)),
        _full((LATENT, N_RADII)),
        _full((1, N_RADII)),
        pl.BlockSpec((BN, 1), lambda i: (i, 0)),
        pl.BlockSpec((BN, 1), lambda i: (i, 0)),
    ],
    out_specs=[
        _full((NUM_GRAPHS, 8)),
        pl.BlockSpec((BN, N_RADII), lambda i: (i, 0)),
    ],
    out_shape=[
        jax.ShapeDtypeStruct((NUM_GRAPHS, 8), _f32),
        jax.ShapeDtypeStruct((N_NODES, N_RADII), _f32),
    ],
)


def kernel(species, senders, receivers, segment_ids, target_species, params):
    spec2 = jnp.pad(species.astype(_i32), (0, N_PAD - N_NODES),
                    constant_values=NUM_ELEMENTS).reshape(N_PAD, 1)
    seg2 = segment_ids.astype(_i32).reshape(N_NODES, 1)
    tsp2 = target_species.astype(_i32).reshape(N_NODES, 1)
    send_g = jnp.pad(senders.astype(_i32), (0, EDGES_PAD - N_EDGES))
    recv_g = jnp.pad(receivers.astype(_i32), (0, EDGES_PAD - N_EDGES))
    recv_p = jnp.pad(receivers.astype(_i32), (0, EDGES_PAD - N_EDGES)).reshape(
        EDGES_PAD // CHUNK, CHUNK)
    zeros = jnp.zeros((ROWS_PER_SUB, LATENT), _f32)
    se = params["species_embed"].astype(_f32)

    centers = jnp.linspace(0.0, 10.0, N_CENTERS).astype(_f32)
    cen_p = jnp.concatenate(
        [centers, jnp.full((LATENT - N_CENTERS,), 1e4, _f32)]).reshape(1, LATENT)

    def row(b):
        return b.astype(_f32).reshape(1, -1)

    _sc_gather, _sc_scatter = _sc_kernels()
    nodes = _embed_init(spec2, se)
    for blk in params["blocks"]:
        (w1, b1), (w2, b2) = blk["cfconv"]
        w1p = jnp.pad(w1.astype(_f32), ((0, LATENT - N_CENTERS), (0, 0)))
        sf, rf = _sc_gather(nodes, send_g, recv_g)
        msgs = _edge_dense(sf, rf, cen_p, w1p, row(b1), w2.astype(_f32),
                           row(b2))
        parts = _sc_scatter(msgs, recv_p, zeros)
        (v1, c1), (v2, c2), (v3, c3) = blk["node"]
        nodes = _node_mlp(parts, v1.astype(_f32), row(c1), v2.astype(_f32),
                          row(c2), v3.astype(_f32), row(c3))

    nodes = nodes[:N_NODES]
    (t1, u1), (t2, u2), (t3, u3) = params["target_mlp"]
    (d1, e1), (d2, e2), (d3, e3) = params["dist_mlp"]
    t3p = jnp.pad(t3.astype(_f32), ((0, 0), (0, 8 - NUM_ELEMENTS)))
    u3p = jnp.pad(u3.astype(_f32), (0, 8 - NUM_ELEMENTS)).reshape(1, 8)
    tslp, dl = _final_heads(
        nodes, se, t1.astype(_f32), row(u1), t2.astype(_f32), row(u2),
        t3p, u3p, d1.astype(_f32), row(e1), d2.astype(_f32), row(e2),
        d3.astype(_f32), row(e3), seg2, tsp2)
    return tslp[:, :NUM_ELEMENTS], dl


# speedup vs baseline: 4.3416x; 1.0971x over previous
import functools

import jax
import jax.numpy as jnp
from jax import lax
from jax.experimental import pallas as pl
from jax.experimental.pallas import tpu as pltpu
from jax.experimental.pallas import tpu_sc as plsc

N_NODES = 10000
N_EDGES = 160000
NUM_GRAPHS = 64
LATENT = 128
NUM_ELEMENTS = 5
N_RADII = 64
N_CENTERS = 100

NC = 2
NS = 16
NW = NC * NS

CHUNK = 128
EDGES_PAD = 163840
CHUNKS_PER_W = EDGES_PAD // (NW * CHUNK)
EDGES_PER_W = EDGES_PAD // NW
GCH = 64
GCHUNKS_PER_W = EDGES_PAD // (NW * GCH)
N_PAD = 10240
ROWS_PER_SUB = N_PAD // NS

BN = 1000
BN2 = 1024
BE = 2048

_f32 = jnp.float32
_i32 = jnp.int32

def _make_gather_body(epw, gchunks):
    def body(nodes_hbm, send_hbm, recv_hbm, sf_hbm, rf_hbm,
             sidx, ridx, sbuf, rbuf, table,
             ws0, ws1, wr0, wr1):
        cid = lax.axis_index("c")
        sid = lax.axis_index("s")
        wid = sid * NC + cid
        pltpu.sync_copy(
            nodes_hbm.at[pl.ds(sid * ROWS_PER_SUB, ROWS_PER_SUB)],
            table.at[pl.ds(sid * ROWS_PER_SUB, ROWS_PER_SUB)])
        base = wid * epw
        pltpu.sync_copy(send_hbm.at[pl.ds(base, epw)], sidx)
        pltpu.sync_copy(recv_hbm.at[pl.ds(base, epw)], ridx)
        plsc.subcore_barrier()
        wsem = (ws0, ws1)
        wrsem = (wr0, wr1)

        def start_write(j, b):
            row = base + j * GCH
            pltpu.async_copy(sbuf.at[b], sf_hbm.at[pl.ds(row, GCH)], wsem[b])
            pltpu.async_copy(rbuf.at[b], rf_hbm.at[pl.ds(row, GCH)], wrsem[b])

        def wait_write(b):
            pltpu.make_async_copy(sbuf.at[b], sf_hbm.at[pl.ds(0, GCH)],
                                  wsem[b]).wait()
            pltpu.make_async_copy(rbuf.at[b], rf_hbm.at[pl.ds(0, GCH)],
                                  wrsem[b]).wait()

        def pair(i, carry):
            for b in (0, 1):
                j = 2 * i + b

                @pl.when(j >= 2)
                def _():
                    wait_write(b)

                pltpu.sync_copy(table.at[sidx.at[pl.ds(j * GCH, GCH)]],
                                sbuf.at[b])
                pltpu.sync_copy(table.at[ridx.at[pl.ds(j * GCH, GCH)]],
                                rbuf.at[b])
                start_write(j, b)
            return carry

        lax.fori_loop(0, gchunks // 2, pair, 0)
        wait_write(0)
        wait_write(1)

    return body


def _make_scatter_body(epw, chunks):
    def body(msg_hbm, recv_hbm, zeros_hbm, out_hbm, ridx, mbuf, acc,
             rs0, rs1):
        cid = lax.axis_index("c")
        sid = lax.axis_index("s")
        wid = sid * NC + cid
        pltpu.sync_copy(zeros_hbm,
                        acc.at[pl.ds(sid * ROWS_PER_SUB, ROWS_PER_SUB)])
        plsc.subcore_barrier()
        pltpu.sync_copy(recv_hbm.at[wid], ridx)
        base = wid * epw
        rsem = (rs0, rs1)

        def start_read(j, b):
            pltpu.async_copy(msg_hbm.at[pl.ds(base + j * CHUNK, CHUNK)],
                             mbuf.at[b], rsem[b])

        def wait_read(b):
            pltpu.make_async_copy(msg_hbm.at[pl.ds(0, CHUNK)], mbuf.at[b],
                                  rsem[b]).wait()

        start_read(0, 0)

        def pair(i, carry):
            for b in (0, 1):
                j = 2 * i + b

                @pl.when(j + 1 < chunks)
                def _():
                    start_read(j + 1, 1 - b)

                wait_read(b)
                pltpu.sync_copy(mbuf.at[b], acc.at[ridx.at[j]], add=True)
            return carry

        lax.fori_loop(0, chunks // 2, pair, 0)
        plsc.subcore_barrier()
        pltpu.sync_copy(
            acc.at[pl.ds(sid * ROWS_PER_SUB, ROWS_PER_SUB)],
            out_hbm.at[cid].at[pl.ds(sid * ROWS_PER_SUB, ROWS_PER_SUB)],
        )

    return body


EDGES_H = EDGES_PAD // 2
EPW_H = EDGES_H // NW
GCHUNKS_H = EPW_H // GCH
CHUNKS_H = EPW_H // CHUNK


@functools.cache
def _sc_kernels():
    mesh = plsc.VectorSubcoreMesh(
        core_axis_name="c", subcore_axis_name="s",
        num_cores=NC, num_subcores=NS)
    gather = pl.kernel(
        _make_gather_body(EPW_H, GCHUNKS_H),
        out_type=(
            jax.ShapeDtypeStruct((EDGES_H, LATENT), _f32),
            jax.ShapeDtypeStruct((EDGES_H, LATENT), _f32),
        ),
        mesh=mesh,
        scratch_types=[
            pltpu.VMEM((EPW_H,), _i32),
            pltpu.VMEM((EPW_H,), _i32),
            pltpu.VMEM((2, GCH, LATENT), _f32),
            pltpu.VMEM((2, GCH, LATENT), _f32),
            pltpu.VMEM_SHARED((N_PAD, LATENT), _f32),
        ] + [pltpu.SemaphoreType.DMA] * 4,
    )
    scatter = pl.kernel(
        _make_scatter_body(EPW_H, CHUNKS_H),
        out_type=jax.ShapeDtypeStruct((NC, N_PAD, LATENT), _f32),
        mesh=mesh,
        scratch_types=[
            pltpu.VMEM((CHUNKS_H, CHUNK), _i32),
            pltpu.VMEM((2, CHUNK, LATENT), _f32),
            pltpu.VMEM_SHARED((N_PAD, LATENT), _f32),
        ] + [pltpu.SemaphoreType.DMA] * 2,
    )
    return gather, scatter


def _ssp(x):
    return (jnp.maximum(x, 0.0) + jnp.log1p(jnp.exp(-jnp.abs(x)))
            - 0.6931471805599453)


def _embed_init_body(spec_ref, se_ref, out_ref):
    spec = spec_ref[...]
    acc = jnp.zeros((BN2, LATENT), _f32)
    for e in range(NUM_ELEMENTS):
        m = (spec == e).astype(_f32)
        acc = acc + m * se_ref[e:e + 1, :]
    out_ref[...] = acc


def _make_edge_body(limit):
    def body(sf_ref, rf_ref, cen_ref, w1_ref, b1_ref, w2_ref, b2_ref,
             out_ref):
        sf = sf_ref[...]
        rf = rf_ref[...]
        diff = sf - rf
        d2 = jnp.sum(diff * diff, axis=1, keepdims=True)
        d = jnp.sqrt(d2 + 1e-12)
        t = d - cen_ref[...]
        emb = jnp.exp(-(t * t))
        h = _ssp(jnp.dot(emb, w1_ref[...], preferred_element_type=_f32)
                 + b1_ref[...])
        filt = (jnp.dot(h, w2_ref[...], preferred_element_type=_f32)
                + b2_ref[...])
        msg = filt * sf
        eid = (pl.program_id(0) * BE
               + lax.broadcasted_iota(_i32, (BE, 1), 0))
        out_ref[...] = jnp.where(eid < limit, msg, 0.0)

    return body


def _node_body(acc_ref, acc2_ref, v1_ref, c1_ref, v2_ref, c2_ref, v3_ref,
               c3_ref, out_ref):
    x = (acc_ref[0] + acc_ref[1]) + (acc2_ref[0] + acc2_ref[1])
    h = _ssp(jnp.dot(x, v1_ref[...], preferred_element_type=_f32)
             + c1_ref[...])
    h = _ssp(jnp.dot(h, v2_ref[...], preferred_element_type=_f32)
             + c2_ref[...])
    out_ref[...] = (jnp.dot(h, v3_ref[...], preferred_element_type=_f32)
                    + c3_ref[...])


def _final_body(x_ref, se_ref, t1_ref, u1_ref, t2_ref, u2_ref, t3_ref,
                u3_ref, d1_ref, e1_ref, d2_ref, e2_ref, d3_ref, e3_ref,
                seg_ref, tsp_ref, tsl_ref, dl_ref):
    i = pl.program_id(0)
    x = x_ref[...]
    per = jnp.zeros((BN, 8), _f32)
    col = lax.broadcasted_iota(_i32, (1, 8), 1)
    for e in range(NUM_ELEMENTS):
        xe = x * se_ref[e:e + 1, :]
        h = _ssp(jnp.dot(xe, t1_ref[...], preferred_element_type=_f32)
                 + u1_ref[...])
        h = _ssp(jnp.dot(h, t2_ref[...], preferred_element_type=_f32)
                 + u2_ref[...])
        pe = jnp.dot(h, t3_ref[...], preferred_element_type=_f32) + u3_ref[...]
        per = per + pe * (col == e).astype(_f32)
    seg = seg_ref[...]
    onehot = (seg == lax.broadcasted_iota(_i32, (BN, NUM_GRAPHS), 1)
              ).astype(_f32)
    part = lax.dot_general(onehot, per, (((0,), (0,)), ((), ())),
                           preferred_element_type=_f32)

    @pl.when(i == 0)
    def _():
        tsl_ref[...] = jnp.zeros_like(tsl_ref)

    tsl_ref[...] += part

    tsp = tsp_ref[...]
    tse = jnp.zeros((BN, LATENT), _f32)
    for e in range(NUM_ELEMENTS):
        m = (tsp == e).astype(_f32)
        tse = tse + m * se_ref[e:e + 1, :]
    y = x * tse
    h = _ssp(jnp.dot(y, d1_ref[...], preferred_element_type=_f32)
             + e1_ref[...])
    h = _ssp(jnp.dot(h, d2_ref[...], preferred_element_type=_f32)
             + e2_ref[...])
    dl_ref[...] = (jnp.dot(h, d3_ref[...], preferred_element_type=_f32)
                   + e3_ref[...])


def _full(shape):
    return pl.BlockSpec(shape, lambda i: tuple(0 for _ in shape))


_embed_init = pl.pallas_call(
    _embed_init_body,
    grid=(N_PAD // BN2,),
    in_specs=[
        pl.BlockSpec((BN2, 1), lambda i: (i, 0)),
        _full((NUM_ELEMENTS, LATENT)),
    ],
    out_specs=pl.BlockSpec((BN2, LATENT), lambda i: (i, 0)),
    out_shape=jax.ShapeDtypeStruct((N_PAD, LATENT), _f32),
)

def _make_edge_dense(limit):
    return pl.pallas_call(
        _make_edge_body(limit),
        grid=(EDGES_H // BE,),
        in_specs=[
            pl.BlockSpec((BE, LATENT), lambda i: (i, 0)),
            pl.BlockSpec((BE, LATENT), lambda i: (i, 0)),
            _full((1, LATENT)),
            _full((LATENT, LATENT)),
            _full((1, LATENT)),
            _full((LATENT, LATENT)),
            _full((1, LATENT)),
        ],
        out_specs=pl.BlockSpec((BE, LATENT), lambda i: (i, 0)),
        out_shape=jax.ShapeDtypeStruct((EDGES_H, LATENT), _f32),
    )


_edge_dense_a = _make_edge_dense(EDGES_H)
_edge_dense_b = _make_edge_dense(N_EDGES - EDGES_H)

_node_mlp = pl.pallas_call(
    _node_body,
    grid=(N_PAD // BN2,),
    in_specs=[
        pl.BlockSpec((NC, BN2, LATENT), lambda i: (0, i, 0)),
        pl.BlockSpec((NC, BN2, LATENT), lambda i: (0, i, 0)),
        _full((LATENT, LATENT)),
        _full((1, LATENT)),
        _full((LATENT, LATENT)),
        _full((1, LATENT)),
        _full((LATENT, LATENT)),
        _full((1, LATENT)),
    ],
    out_specs=pl.BlockSpec((BN2, LATENT), lambda i: (i, 0)),
    out_shape=jax.ShapeDtypeStruct((N_PAD, LATENT), _f32),
)

_final_heads = pl.pallas_call(
    _final_body,
    grid=(N_NODES // BN,),
    in_specs=[
        pl.BlockSpec((BN, LATENT), lambda i: (i, 0)),
        _full((NUM_ELEMENTS, LATENT)),
        _full((LATENT, LATENT)),
        _full((1, LATENT)),
        _full((LATENT, LATENT)),
        _full((1, LATENT)),
        _full((LATENT, 8)),
        _full((1, 8)),
        _full((LATENT, LATENT)),
        _full((1, LATENT)),
        _full((LATENT, LATENT)),
        _full((1, LATENT)),
        _full((LATENT, N_RADII)),
        _full((1, N_RADII)),
        pl.BlockSpec((BN, 1), lambda i: (i, 0)),
        pl.BlockSpec((BN, 1), lambda i: (i, 0)),
    ],
    out_specs=[
        _full((NUM_GRAPHS, 8)),
        pl.BlockSpec((BN, N_RADII), lambda i: (i, 0)),
    ],
    out_shape=[
        jax.ShapeDtypeStruct((NUM_GRAPHS, 8), _f32),
        jax.ShapeDtypeStruct((N_NODES, N_RADII), _f32),
    ],
)


def kernel(species, senders, receivers, segment_ids, target_species, params):
    spec2 = jnp.pad(species.astype(_i32), (0, N_PAD - N_NODES),
                    constant_values=NUM_ELEMENTS).reshape(N_PAD, 1)
    seg2 = segment_ids.astype(_i32).reshape(N_NODES, 1)
    tsp2 = target_species.astype(_i32).reshape(N_NODES, 1)
    send_g = jnp.pad(senders.astype(_i32), (0, EDGES_PAD - N_EDGES))
    recv_g = jnp.pad(receivers.astype(_i32), (0, EDGES_PAD - N_EDGES))
    recv_p = jnp.pad(receivers.astype(_i32), (0, EDGES_PAD - N_EDGES)).reshape(
        EDGES_PAD // CHUNK, CHUNK)
    zeros = jnp.zeros((ROWS_PER_SUB, LATENT), _f32)
    se = params["species_embed"].astype(_f32)

    centers = jnp.linspace(0.0, 10.0, N_CENTERS).astype(_f32)
    cen_p = jnp.concatenate(
        [centers, jnp.full((LATENT - N_CENTERS,), 1e4, _f32)]).reshape(1, LATENT)

    def row(b):
        return b.astype(_f32).reshape(1, -1)

    _sc_gather, _sc_scatter = _sc_kernels()
    send_a, send_b = send_g[:EDGES_H], send_g[EDGES_H:]
    recv_a, recv_b = recv_g[:EDGES_H], recv_g[EDGES_H:]
    recv_pa = recv_p[:EDGES_H // CHUNK].reshape(NW, CHUNKS_H, CHUNK)
    recv_pb = recv_p[EDGES_H // CHUNK:].reshape(NW, CHUNKS_H, CHUNK)
    nodes = _embed_init(spec2, se)
    for blk in params["blocks"]:
        (w1, b1), (w2, b2) = blk["cfconv"]
        w1p = jnp.pad(w1.astype(_f32), ((0, LATENT - N_CENTERS), (0, 0)))
        w2f, b2r, b1r = w2.astype(_f32), row(b2), row(b1)
        sfa, rfa = _sc_gather(nodes, send_a, recv_a)
        sfb, rfb = _sc_gather(nodes, send_b, recv_b)
        msga = _edge_dense_a(sfa, rfa, cen_p, w1p, b1r, w2f, b2r)
        msgb = _edge_dense_b(sfb, rfb, cen_p, w1p, b1r, w2f, b2r)
        pa = _sc_scatter(msga, recv_pa, zeros)
        pb = _sc_scatter(msgb, recv_pb, zeros)
        (v1, c1), (v2, c2), (v3, c3) = blk["node"]
        nodes = _node_mlp(pa, pb, v1.astype(_f32), row(c1), v2.astype(_f32),
                          row(c2), v3.astype(_f32), row(c3))

    nodes = nodes[:N_NODES]
    (t1, u1), (t2, u2), (t3, u3) = params["target_mlp"]
    (d1, e1), (d2, e2), (d3, e3) = params["dist_mlp"]
    t3p = jnp.pad(t3.astype(_f32), ((0, 0), (0, 8 - NUM_ELEMENTS)))
    u3p = jnp.pad(u3.astype(_f32), (0, 8 - NUM_ELEMENTS)).reshape(1, 8)
    tslp, dl = _final_heads(
        nodes, se, t1.astype(_f32), row(u1), t2.astype(_f32), row(u2),
        t3p, u3p, d1.astype(_f32), row(e1), d2.astype(_f32), row(e2),
        d3.astype(_f32), row(e3), seg2, tsp2)
    return tslp[:, :NUM_ELEMENTS], dl
